# Initial kernel scaffold; baseline (speedup 1.0000x reference)
import functools

import jax
import jax.numpy as jnp
from jax.experimental import pallas as pl
from jax.experimental.pallas import tpu as pltpu

N_EXPERTS = 8
D_MODEL = 1024
N_TOKENS = 4096
N_BITS = 3
TB = 1024


def _body(z_ref, rp_ref, w_ref, b_ref, out_ref, idx_scr):
    e = pl.program_id(1)

    @pl.when(e == 0)
    def _():
        s = jnp.dot(z_ref[...], rp_ref[...], preferred_element_type=jnp.float32)
        lane = jax.lax.broadcasted_iota(jnp.int32, (TB, 128), 1)
        pw = jnp.where(lane < N_BITS, jnp.left_shift(1, lane), 0)
        bits = jnp.where(s > 0.0, pw, 0)
        idx_scr[...] = jnp.broadcast_to(
            jnp.sum(bits, axis=1, keepdims=True), (TB, 128)
        )

    mask = idx_scr[:, 0:1] == e
    y = jnp.dot(
        z_ref[...].astype(jnp.bfloat16),
        w_ref[0].astype(jnp.bfloat16),
        preferred_element_type=jnp.float32,
    ) + b_ref[...]
    contrib = jnp.where(mask, y, 0.0)

    @pl.when(e == 0)
    def _():
        out_ref[...] = contrib

    @pl.when(e > 0)
    def _():
        out_ref[...] += contrib


@jax.jit
def kernel(z, R, W, b):
    rp = jnp.zeros((D_MODEL, 128), jnp.float32).at[:, :N_BITS].set(R)
    grid = (N_TOKENS // TB, N_EXPERTS)
    return pl.pallas_call(
        _body,
        grid=grid,
        in_specs=[
            pl.BlockSpec((TB, D_MODEL), lambda t, e: (t, 0)),
            pl.BlockSpec((D_MODEL, 128), lambda t, e: (0, 0)),
            pl.BlockSpec((1, D_MODEL, D_MODEL), lambda t, e: (e, 0, 0)),
            pl.BlockSpec((1, D_MODEL), lambda t, e: (e, 0)),
        ],
        out_specs=pl.BlockSpec((TB, D_MODEL), lambda t, e: (t, 0)),
        out_shape=jax.ShapeDtypeStruct((N_TOKENS, D_MODEL), jnp.float32),
        scratch_shapes=[pltpu.VMEM((TB, 128), jnp.int32)],
    )(z, rp, W, b)


# speedup vs baseline: 1.7293x; 1.7293x over previous
import functools

import jax
import jax.numpy as jnp
from jax.experimental import pallas as pl
from jax.experimental.pallas import tpu as pltpu

N_EXPERTS = 8
D_MODEL = 1024
N_TOKENS = 4096
N_BITS = 3
TB = 1024


def _body(z_ref, rp_ref, w_ref, b_ref, out_ref, idx_scr):
    e = pl.program_id(1)

    @pl.when(e == 0)
    def _():
        s = jnp.dot(z_ref[...], rp_ref[...], preferred_element_type=jnp.float32)
        lane = jax.lax.broadcasted_iota(jnp.int32, (TB, 128), 1)
        pw = jnp.where(lane < N_BITS, jnp.left_shift(1, lane), 0)
        bits = jnp.where(s > 0.0, pw, 0)
        idx_scr[...] = jnp.broadcast_to(
            jnp.sum(bits, axis=1, keepdims=True), (TB, 128)
        )

    mask = idx_scr[:, 0:1] == e
    y = jnp.dot(
        z_ref[...].astype(jnp.bfloat16),
        w_ref[0].astype(jnp.bfloat16),
        preferred_element_type=jnp.float32,
    ) + b_ref[0]
    contrib = jnp.where(mask, y, 0.0)

    @pl.when(e == 0)
    def _():
        out_ref[...] = contrib

    @pl.when(e > 0)
    def _():
        out_ref[...] += contrib


@jax.jit
def kernel(z, R, W, b):
    rp = jnp.zeros((D_MODEL, 128), jnp.float32).at[:, :N_BITS].set(R)
    grid = (N_TOKENS // TB, N_EXPERTS)
    return pl.pallas_call(
        _body,
        grid=grid,
        in_specs=[
            pl.BlockSpec((TB, D_MODEL), lambda t, e: (t, 0)),
            pl.BlockSpec((D_MODEL, 128), lambda t, e: (0, 0)),
            pl.BlockSpec((1, D_MODEL, D_MODEL), lambda t, e: (e, 0, 0)),
            pl.BlockSpec((1, 1, D_MODEL), lambda t, e: (e, 0, 0)),
        ],
        out_specs=pl.BlockSpec((TB, D_MODEL), lambda t, e: (t, 0)),
        out_shape=jax.ShapeDtypeStruct((N_TOKENS, D_MODEL), jnp.float32),
        scratch_shapes=[pltpu.VMEM((TB, 128), jnp.int32)],
    )(z, rp, W, b.reshape(N_EXPERTS, 1, D_MODEL))


# speedup vs baseline: 1.7793x; 1.0290x over previous
import functools

import jax
import jax.numpy as jnp
from jax import lax
from jax.experimental import pallas as pl
from jax.experimental.pallas import tpu as pltpu
from jax.experimental.pallas import tpu_sc as plsc

N_EXPERTS = 8
D = 1024
N = 4096
N_BITS = 3

BM = 256
NB = N // BM
T = NB + N_EXPERTS - 1

NW = 32
RPW = N // NW
SB = 64
NSB = RPW // SB


def _route_body(z_ref, rp_ref, dest_ref, cnt_ref, work):
    xdot = functools.partial(jnp.dot, precision=jax.lax.Precision.HIGHEST,
                             preferred_element_type=jnp.float32)
    s = jnp.dot(z_ref[...], rp_ref[...], preferred_element_type=jnp.float32)
    lane = lax.broadcasted_iota(jnp.int32, (N, 128), 1)
    row = lax.broadcasted_iota(jnp.int32, (N, 128), 0)
    pw = jnp.where(lane < N_BITS, jnp.left_shift(1, lane), 0)
    idxv = jnp.sum(jnp.where(s > 0.0, pw, 0), axis=1, keepdims=True)
    onehot = (lane == idxv).astype(jnp.float32)

    g_r = lax.broadcasted_iota(jnp.int32, (32, N), 0)
    t_c = lax.broadcasted_iota(jnp.int32, (32, N), 1)
    gmat = (t_c // 128 == g_r).astype(jnp.float32)
    gsum = jnp.dot(gmat, onehot, preferred_element_type=jnp.float32)
    r32 = lax.broadcasted_iota(jnp.int32, (32, 32), 0)
    c32 = lax.broadcasted_iota(jnp.int32, (32, 32), 1)
    gpre = xdot((c32 < r32).astype(jnp.float32), gsum)
    r_g = lax.broadcasted_iota(jnp.int32, (N, 32), 0)
    c_g = lax.broadcasted_iota(jnp.int32, (N, 32), 1)
    gsel = (r_g // 128 == c_g).astype(jnp.float32)
    pre_full = xdot(gsel, gpre)

    r128 = lax.broadcasted_iota(jnp.int32, (128, 128), 0)
    c128 = lax.broadcasted_iota(jnp.int32, (128, 128), 1)
    tril = (c128 < r128).astype(jnp.float32)
    for g in range(32):
        work[g * 128:(g + 1) * 128, :] = jnp.dot(
            tril, onehot[g * 128:(g + 1) * 128, :],
            preferred_element_type=jnp.float32)
    rank = jnp.sum(onehot * (work[...] + pre_full), axis=1, keepdims=True)

    counts = jnp.dot(jnp.ones((1, 32), jnp.float32), gsum,
                     preferred_element_type=jnp.float32)
    starts_row = xdot(counts, (r128 < c128).astype(jnp.float32))
    start_sel = jnp.sum(onehot * starts_row, axis=1, keepdims=True)
    dest = rank + start_sel

    lsel = (lane == row % 128).astype(jnp.float32)
    dest_ref[...] = xdot(gmat, dest * lsel).astype(jnp.int32)
    cnt_ref[...] = counts.astype(jnp.int32)


def _route(z, rp):
    return pl.pallas_call(
        _route_body,
        in_specs=[
            pl.BlockSpec((N, D), lambda: (0, 0)),
            pl.BlockSpec((D, 128), lambda: (0, 0)),
        ],
        out_specs=[
            pl.BlockSpec((32, 128), lambda: (0, 0)),
            pl.BlockSpec((1, 128), lambda: (0, 0)),
        ],
        out_shape=[
            jax.ShapeDtypeStruct((32, 128), jnp.int32),
            jax.ShapeDtypeStruct((1, 128), jnp.int32),
        ],
        scratch_shapes=[pltpu.VMEM((N, 128), jnp.float32)],
    )(z, rp)


def _work_lists(counts):
    starts9 = jnp.concatenate(
        [jnp.zeros((1,), jnp.int32), jnp.cumsum(counts)]).astype(jnp.int32)
    lo, hi = starts9[:-1], starts9[1:]
    bkr = jnp.arange(NB, dtype=jnp.int32)[None, :]
    ov = (hi[:, None] > bkr * BM) & (lo[:, None] < (bkr + 1) * BM) \
        & (hi[:, None] > lo[:, None])
    ovf = ov.reshape(-1)
    pos = jnp.cumsum(ovf.astype(jnp.int32)) - 1
    nreal = jnp.sum(ovf.astype(jnp.int32))
    tgt = jnp.where(ovf, pos, T)
    e_flat = jnp.broadcast_to(
        jnp.arange(N_EXPERTS, dtype=jnp.int32)[:, None], (N_EXPERTS, NB)
    ).reshape(-1)
    bk_flat = jnp.broadcast_to(bkr, (N_EXPERTS, NB)).reshape(-1)
    first_flat = (ov & (jnp.cumsum(ov.astype(jnp.int32), axis=0) == 1)) \
        .astype(jnp.int32).reshape(-1)
    blk_sc = jnp.zeros((T + 1,), jnp.int32).at[tgt].set(bk_flat, mode="drop")[:T]
    exp_sc = jnp.zeros((T + 1,), jnp.int32).at[tgt].set(e_flat, mode="drop")[:T]
    fst_sc = jnp.zeros((T + 1,), jnp.int32).at[tgt].set(first_flat, mode="drop")[:T]
    slot = jnp.arange(T, dtype=jnp.int32)
    valid = (slot < nreal).astype(jnp.int32)
    blk = jnp.where(slot < nreal, blk_sc, NB - 1)
    last_e = jnp.max(jnp.where(counts > 0, jnp.arange(N_EXPERTS, dtype=jnp.int32), 0))
    exp = jnp.where(slot < nreal, exp_sc, last_e)
    return starts9, blk, exp, valid, fst_sc


def _gmm_body(starts_s, blk_s, exp_s, valid_s, first_s,
              zs_ref, w_ref, b_ref, out_ref):
    t = pl.program_id(0)
    e = exp_s[t]
    lo = starts_s[e]
    hi = starts_s[e + 1]
    rows = blk_s[t] * BM + lax.broadcasted_iota(jnp.int32, (BM, 1), 0)
    mask = (rows >= lo) & (rows < hi) & (valid_s[t] > 0)
    y = jnp.dot(
        zs_ref[...].astype(jnp.bfloat16),
        w_ref[0].astype(jnp.bfloat16),
        preferred_element_type=jnp.float32,
    ) + b_ref[0]
    contrib = jnp.where(mask, y, 0.0)

    @pl.when(first_s[t] == 1)
    def _():
        out_ref[...] = contrib

    @pl.when(first_s[t] == 0)
    def _():
        out_ref[...] += contrib


def _gmm(starts9, blk, exp, valid, first, zs, W, b3):
    grid_spec = pltpu.PrefetchScalarGridSpec(
        num_scalar_prefetch=5,
        grid=(T,),
        in_specs=[
            pl.BlockSpec((BM, D), lambda t, st, bk, ex, va, fi: (bk[t], 0)),
            pl.BlockSpec((1, D, D), lambda t, st, bk, ex, va, fi: (ex[t], 0, 0)),
            pl.BlockSpec((1, 1, D), lambda t, st, bk, ex, va, fi: (ex[t], 0, 0)),
        ],
        out_specs=pl.BlockSpec((BM, D), lambda t, st, bk, ex, va, fi: (bk[t], 0)),
    )
    return pl.pallas_call(
        _gmm_body,
        grid_spec=grid_spec,
        out_shape=jax.ShapeDtypeStruct((N, D), jnp.float32),
    )(starts9, blk, exp, valid, first, zs, W, b3)


@functools.lru_cache(maxsize=1)
def _sc_kernels():
    mesh = plsc.VectorSubcoreMesh(core_axis_name="c", subcore_axis_name="s")
    deco = functools.partial(
        pl.kernel,
        mesh=mesh,
        out_type=jax.ShapeDtypeStruct((N, D), jnp.float32),
        scratch_types=[
            pltpu.VMEM((RPW,), jnp.int32),
            pltpu.VMEM((SB,), jnp.int32),
            pltpu.VMEM((SB, D), jnp.float32),
            pltpu.SemaphoreType.DMA,
        ],
    )

    def load_batch(idx_all, idx_b, j):
        for k in range(SB // 16):
            idx_b[k * 16:(k + 1) * 16] = idx_all[j * SB + k * 16:
                                                 j * SB + (k + 1) * 16]

    @deco
    def sc_dispatch(z_hbm, d2_hbm, zs_hbm, idx_all, idx_b, rows_v, sem):
        wid = lax.axis_index("s") * 2 + lax.axis_index("c")
        base = wid * RPW
        pltpu.sync_copy(d2_hbm.at[wid], idx_all)
        for j in range(NSB):
            load_batch(idx_all, idx_b, j)
            pltpu.sync_copy(z_hbm.at[pl.ds(base + j * SB, SB)], rows_v)
            pltpu.async_copy(rows_v, zs_hbm.at[idx_b], sem).wait()

    @deco
    def sc_combine(ys_hbm, d2_hbm, out_hbm, idx_all, idx_b, rows_v, sem):
        wid = lax.axis_index("s") * 2 + lax.axis_index("c")
        base = wid * RPW
        pltpu.sync_copy(d2_hbm.at[wid], idx_all)
        for j in range(NSB):
            load_batch(idx_all, idx_b, j)
            pltpu.async_copy(ys_hbm.at[idx_b], rows_v, sem).wait()
            pltpu.sync_copy(rows_v, out_hbm.at[pl.ds(base + j * SB, SB)])

    return sc_dispatch, sc_combine


@jax.jit
def kernel(z, R, W, b):
    rp = jnp.zeros((D, 128), jnp.float32).at[:, :N_BITS].set(R)
    dest2d, cnts = _route(z, rp)
    counts = cnts[0, :N_EXPERTS]
    starts9, blk, exp, valid, first = _work_lists(counts)
    sc_dispatch, sc_combine = _sc_kernels()
    zs = sc_dispatch(z, dest2d)
    ys = _gmm(starts9, blk, exp, valid, first, zs, W,
              b.reshape(N_EXPERTS, 1, D))
    return sc_combine(ys, dest2d)
